# baseline (speedup 1.0000x reference)
import functools

import jax
import jax.numpy as jnp
from jax import lax
from jax.experimental import pallas as pl
from jax.experimental.pallas import tpu as pltpu
from jax.experimental.pallas import tpu_sc as plsc

_B = 16384
_D = 32
_NC = 2
_NS = 16
_NW = _NC * _NS
_BPW = _B // _NW
_NCHUNK = _BPW // 16


def _sc_score(E_hbm, rv_hbm, u_hbm, r_hbm, v_hbm, out_hbm,
              u_idx_v, r_idx_v, v_idx_v,
              u_rows, v_rows, r_rows, out_v,
              sem_u, sem_v, sem_r):
    wid = lax.axis_index("s") * _NC + lax.axis_index("c")
    base = wid * _BPW

    pltpu.sync_copy(u_hbm.at[pl.ds(base, _BPW)], u_idx_v)
    pltpu.sync_copy(v_hbm.at[pl.ds(base, _BPW)], v_idx_v)
    pltpu.sync_copy(r_hbm.at[pl.ds(base, _BPW)], r_idx_v)

    cu = pltpu.async_copy(E_hbm.at[u_idx_v], u_rows, sem_u)
    cv = pltpu.async_copy(E_hbm.at[v_idx_v], v_rows, sem_v)
    cr = pltpu.async_copy(rv_hbm.at[r_idx_v], r_rows, sem_r)
    cu.wait()
    cv.wait()
    cr.wait()

    lanes = lax.iota(jnp.int32, 16)

    def chunk_body(c, carry):
        b0 = c * 16
        rows = b0 + lanes
        acc = jnp.zeros((16,), jnp.float32)
        for d in range(_D):
            col = jnp.full((16,), d, jnp.int32)
            ud = plsc.load_gather(u_rows, [rows, col])
            vd = plsc.load_gather(v_rows, [rows, col])
            rd = plsc.load_gather(r_rows, [rows, col])
            t = ud - (vd + rd)
            acc = acc + t * t
        out_v[pl.ds(b0, 16)] = -acc
        return carry

    lax.fori_loop(0, _NCHUNK, chunk_body, 0)

    pltpu.sync_copy(out_v, out_hbm.at[pl.ds(base, _BPW)])


@jax.jit
def kernel(E, rv, u_idx, r_idx, v_idx):
    mesh = plsc.VectorSubcoreMesh(core_axis_name="c", subcore_axis_name="s")
    run = pl.kernel(
        _sc_score,
        out_type=jax.ShapeDtypeStruct((_B,), jnp.float32),
        mesh=mesh,
        compiler_params=pltpu.CompilerParams(
            needs_layout_passes=False, use_tc_tiling_on_sc=False
        ),
        scratch_types=[
            pltpu.VMEM((_BPW,), jnp.int32),
            pltpu.VMEM((_BPW,), jnp.int32),
            pltpu.VMEM((_BPW,), jnp.int32),
            pltpu.VMEM((_BPW, _D), jnp.float32),
            pltpu.VMEM((_BPW, _D), jnp.float32),
            pltpu.VMEM((_BPW, _D), jnp.float32),
            pltpu.VMEM((_BPW,), jnp.float32),
            pltpu.SemaphoreType.DMA,
            pltpu.SemaphoreType.DMA,
            pltpu.SemaphoreType.DMA,
        ],
    )
    return run(E, rv, u_idx, r_idx, v_idx)
